# baseline (speedup 1.0000x reference)
import jax
import jax.numpy as jnp
from jax import lax
from jax.experimental import pallas as pl
from jax.experimental.pallas import tpu as pltpu
from jax.experimental.pallas import tpu_sc as plsc

VOCAB = 100000
D_MODEL = 1024
BATCH = 4
SEQ = 2048
N_TOKENS = 20

_NC = 2
_NS = 16
_NW = _NC * _NS
_ROWS = BATCH * SEQ
_RPW = _ROWS // _NW
_CH = 32
_NCHUNK = _RPW // _CH
_WPB = SEQ // _RPW
_PSTRIDE = 32
_PN = 24
_NBUF = 2


def _body(tokens_hbm, wte_hbm, learned_hbm, pidx_hbm, out_hbm,
          tok_v, idx_v, rows0_v, rows1_v, learned_v, pidx_v,
          gsem0, gsem1, ssem0, ssem1, psem):
    wid = lax.axis_index("s") * _NC + lax.axis_index("c")
    base = wid * _RPW
    owns_prompt = wid % _WPB == 0
    b = wid // _WPB

    pltpu.sync_copy(tokens_hbm, tok_v)
    p0 = (wid % _WPB) * _RPW
    for i in range(_RPW // 16):
        idx_v[pl.ds(i * 16, 16)] = tok_v[b, pl.ds(p0 + i * 16, 16)]

    bufs = (rows0_v, rows1_v)
    gsems = (gsem0, gsem1)
    ssems = (ssem0, ssem1)

    def gather_desc(j, n):
        return pltpu.make_async_copy(
            wte_hbm.at[idx_v.at[pl.ds(j * _CH, _CH)]], bufs[n], gsems[n])

    def scatter_desc(j, n):
        return pltpu.make_async_copy(
            bufs[n], out_hbm.at[pl.ds(base + j * _CH, _CH)], ssems[n])

    for n in range(_NBUF):
        gather_desc(n, n).start()

    def ring_iter(k, carry):
        for n in range(_NBUF):
            j = k * _NBUF + n
            gather_desc(j, n).wait()
            scatter_desc(j, n).start()

            @pl.when(j + _NBUF < _NCHUNK)
            def _():
                scatter_desc(j, n).wait()
                gather_desc(j + _NBUF, n).start()
        return carry

    lax.fori_loop(0, _NCHUNK // _NBUF, ring_iter, 0)
    for n in range(_NBUF):
        scatter_desc(_NCHUNK - _NBUF + n, n).wait()

    @pl.when(owns_prompt)
    def _():
        pltpu.sync_copy(pidx_hbm.at[pl.ds(b * _PSTRIDE, _PN)], pidx_v)
        pltpu.sync_copy(learned_hbm, learned_v)
        pltpu.async_copy(learned_v, out_hbm.at[pidx_v], psem).wait()


@jax.jit
def _soft_embedding(tokens, wte_weight, learned_embedding):
    t = jnp.arange(_PSTRIDE, dtype=jnp.int32) % N_TOKENS
    pidx = (jnp.arange(BATCH, dtype=jnp.int32)[:, None] * SEQ + 1 + t[None, :]
            ).reshape(BATCH * _PSTRIDE)
    learned_pad = learned_embedding[
        jnp.arange(_PN, dtype=jnp.int32) % N_TOKENS
    ]
    mesh = plsc.VectorSubcoreMesh(core_axis_name="c", subcore_axis_name="s")
    out = pl.kernel(
        _body,
        out_type=jax.ShapeDtypeStruct((_ROWS, D_MODEL), jnp.float32),
        mesh=mesh,
        scratch_types=[
            pltpu.VMEM((BATCH, SEQ), jnp.int32),
            pltpu.VMEM((_RPW,), jnp.int32),
            pltpu.VMEM((_CH, D_MODEL), jnp.float32),
            pltpu.VMEM((_CH, D_MODEL), jnp.float32),
            pltpu.VMEM((_PN, D_MODEL), jnp.float32),
            pltpu.VMEM((_PN,), jnp.int32),
            pltpu.SemaphoreType.DMA,
            pltpu.SemaphoreType.DMA,
            pltpu.SemaphoreType.DMA,
            pltpu.SemaphoreType.DMA,
            pltpu.SemaphoreType.DMA,
        ],
    )(tokens, wte_weight, learned_pad, pidx)
    return out.reshape(BATCH, SEQ, D_MODEL)


def kernel(tokens, wte_weight, learned_embedding):
    return _soft_embedding(tokens, wte_weight, learned_embedding)


# speedup vs baseline: 1.0614x; 1.0614x over previous
import jax
import jax.numpy as jnp
from jax import lax
from jax.experimental import pallas as pl
from jax.experimental.pallas import tpu as pltpu
from jax.experimental.pallas import tpu_sc as plsc

VOCAB = 100000
D_MODEL = 1024
BATCH = 4
SEQ = 2048
N_TOKENS = 20

_NC = 2
_NS = 16
_NW = _NC * _NS
_ROWS = BATCH * SEQ
_RPW = _ROWS // _NW
_CH = 32
_NCHUNK = _RPW // _CH
_WPB = SEQ // _RPW
_PSTRIDE = 32
_PN = 24
_NBUF = 2
_CHUNKS = ((0, 64), (64, 56), (120, 64), (184, 56), (240, 16))


def _body(idx_hbm, wte_hbm, learned_hbm, pidx_hbm, out_hbm,
          idx_v, rows0_v, rows1_v, pidx_v,
          gsem0, gsem1, ssem0, ssem1, psem):
    wid = lax.axis_index("s") * _NC + lax.axis_index("c")
    base = wid * _RPW
    owns_prompt = wid % _WPB == 0
    b = wid // _WPB

    pltpu.sync_copy(idx_hbm.at[pl.ds(base, _RPW)], idx_v)

    bufs = (rows0_v, rows1_v)
    gsems = (gsem0, gsem1)
    ssems = (ssem0, ssem1)

    def gather_desc(j):
        off, sz = _CHUNKS[j]
        n = j % _NBUF
        return pltpu.make_async_copy(
            wte_hbm.at[idx_v.at[pl.ds(off, sz)]], bufs[n].at[pl.ds(0, sz)],
            gsems[n])

    def scatter_desc(j):
        off, sz = _CHUNKS[j]
        n = j % _NBUF
        return pltpu.make_async_copy(
            bufs[n].at[pl.ds(0, sz)], out_hbm.at[pl.ds(base + off, sz)],
            ssems[n])

    nch = len(_CHUNKS)
    for j in range(_NBUF):
        gather_desc(j).start()
    for j in range(nch):
        gather_desc(j).wait()
        scatter_desc(j).start()
        if j + _NBUF < nch:
            scatter_desc(j).wait()
            gather_desc(j + _NBUF).start()
    for j in range(max(0, nch - _NBUF), nch):
        scatter_desc(j).wait()

    @pl.when(owns_prompt)
    def _():
        pltpu.sync_copy(pidx_hbm.at[pl.ds(b * _PSTRIDE, _PN)], pidx_v)
        pltpu.sync_copy(learned_hbm, rows0_v.at[pl.ds(0, _PN)])
        pltpu.async_copy(rows0_v.at[pl.ds(0, _PN)], out_hbm.at[pidx_v],
                         psem).wait()


@jax.jit
def _soft_embedding(tokens, wte_weight, learned_embedding):
    idx = tokens.reshape(_ROWS)
    t = jnp.arange(_PSTRIDE, dtype=jnp.int32) % N_TOKENS
    pidx = (jnp.arange(BATCH, dtype=jnp.int32)[:, None] * SEQ + 1 + t[None, :]
            ).reshape(BATCH * _PSTRIDE)
    learned_pad = learned_embedding[
        jnp.arange(_PN, dtype=jnp.int32) % N_TOKENS
    ]
    mesh = plsc.VectorSubcoreMesh(core_axis_name="c", subcore_axis_name="s")
    out = pl.kernel(
        _body,
        out_type=jax.ShapeDtypeStruct((_ROWS, D_MODEL), jnp.float32),
        mesh=mesh,
        scratch_types=[
            pltpu.VMEM((_RPW,), jnp.int32),
            pltpu.VMEM((_CHUNKS[0][1], D_MODEL), jnp.float32),
            pltpu.VMEM((_CHUNKS[1][1], D_MODEL), jnp.float32),
            pltpu.VMEM((_PN,), jnp.int32),
            pltpu.SemaphoreType.DMA,
            pltpu.SemaphoreType.DMA,
            pltpu.SemaphoreType.DMA,
            pltpu.SemaphoreType.DMA,
            pltpu.SemaphoreType.DMA,
        ],
    )(idx, wte_weight, learned_pad, pidx)
    return out.reshape(BATCH, SEQ, D_MODEL)


def kernel(tokens, wte_weight, learned_embedding):
    return _soft_embedding(tokens, wte_weight, learned_embedding)
